# baseline (speedup 1.0000x reference)
import functools
import jax
import jax.numpy as jnp
from jax import lax
from jax.experimental import pallas as pl
from jax.experimental.pallas import tpu as pltpu
from jax.experimental.pallas import tpu_sc as plsc

VOCAB = 30522
HIDDEN = 768
MAX_POS = 512
TYPE_VOCAB = 16
BATCH = 128
SEQ = 512

L = 16
NW = 32
PBLK = SEQ // NW
SCH = 4
CHT = SCH * PBLK
NCHUNK = BATCH // SCH
NG = HIDDEN // L
NQ = CHT // 4
INV_H = 1.0 / HIDDEN
EPS = 1e-12


def _rsqrt(x):
    xi = plsc.bitcast(x, jnp.int32)
    yi = jnp.int32(0x5F3759DF) - (xi >> 1)
    y = plsc.bitcast(yi, jnp.float32)
    for _ in range(3):
        y = y * (1.5 - 0.5 * x * y * y)
    return y


def _body(ids_hbm, tt_hbm, word_hbm, pos_hbm, type_hbm, gamma_hbm, beta_hbm,
          out_hbm, idx_v, tt_v, rows_v, pos_v, type_v, gam_v, bet_v, sem):
    wid = lax.axis_index("s") * 2 + lax.axis_index("c")
    p0 = wid * PBLK

    pltpu.sync_copy(pos_hbm.at[pl.ds(p0, PBLK), :], pos_v)
    pltpu.sync_copy(type_hbm, type_v)
    pltpu.sync_copy(gamma_hbm, gam_v)
    pltpu.sync_copy(beta_hbm, bet_v)

    zero = jnp.zeros((L,), jnp.float32)

    def chunk_body(c, _):
        s0 = c * SCH
        for g in range(SCH):
            pltpu.sync_copy(ids_hbm.at[s0 + g, pl.ds(p0, PBLK)],
                            idx_v.at[pl.ds(g * PBLK, PBLK)])
            pltpu.sync_copy(tt_hbm.at[s0 + g, pl.ds(p0, PBLK)],
                            tt_v.at[pl.ds(g * PBLK, PBLK)])
        pltpu.async_copy(word_hbm.at[idx_v], rows_v, sem).wait()

        def group_body(g2, _):
            tb = g2 * L
            tt16 = tt_v[pl.ds(tb, L)]
            for q in range(4):
                mv = []
                rv = []
                for k in range(4):
                    kk = q * 4 + k
                    t = tb + kk
                    tid = tt16[kk]

                    def p1(i, carry, t=t, kk=kk, tid=tid):
                        sm, sq = carry
                        base = i * (24 * L)
                        for u in range(24):
                            sl = pl.ds(base + u * L, L)
                            v = rows_v[t, sl] + pos_v[kk, sl] + type_v[tid, sl]
                            rows_v[t, sl] = v
                            sm = sm + v
                            sq = sq + v * v
                        return sm, sq

                    sm, sq = lax.fori_loop(0, 2, p1, (zero, zero))
                    mean = jnp.sum(sm) * INV_H
                    var = jnp.sum(sq) * INV_H - mean * mean
                    mv.append(jnp.full((L,), mean, jnp.float32))
                    rv.append(_rsqrt(jnp.full((L,), var + EPS, jnp.float32)))

                def p2(i, _):
                    base = i * (4 * L)
                    for u in range(4):
                        sl = pl.ds(base + u * L, L)
                        ga = gam_v[sl]
                        be = bet_v[sl]
                        for k in range(4):
                            t = tb + q * 4 + k
                            v = (rows_v[t, sl] - mv[k]) * rv[k]
                            rows_v[t, sl] = v * ga + be
                    return 0

                lax.fori_loop(0, NG // 4, p2, 0)
            return 0

        lax.fori_loop(0, SCH, group_body, 0)

        for g in range(SCH):
            pltpu.sync_copy(rows_v.at[pl.ds(g * PBLK, PBLK), :],
                            out_hbm.at[s0 + g, pl.ds(p0, PBLK), :])
        return 0

    lax.fori_loop(0, NCHUNK, chunk_body, 0)


@jax.jit
def _run(input_ids, token_type_ids, word_table, pos_table, type_table,
         gamma, beta):
    mesh = plsc.VectorSubcoreMesh(core_axis_name="c", subcore_axis_name="s")
    f = pl.kernel(
        _body,
        out_type=jax.ShapeDtypeStruct((BATCH, SEQ, HIDDEN), jnp.float32),
        mesh=mesh,
        compiler_params=pltpu.CompilerParams(needs_layout_passes=False),
        scratch_types=[
            pltpu.VMEM((CHT,), jnp.int32),
            pltpu.VMEM((CHT,), jnp.int32),
            pltpu.VMEM((CHT, HIDDEN), jnp.float32),
            pltpu.VMEM((PBLK, HIDDEN), jnp.float32),
            pltpu.VMEM((TYPE_VOCAB, HIDDEN), jnp.float32),
            pltpu.VMEM((HIDDEN,), jnp.float32),
            pltpu.VMEM((HIDDEN,), jnp.float32),
            pltpu.SemaphoreType.DMA,
        ],
    )
    return f(input_ids, token_type_ids, word_table, pos_table, type_table,
             gamma, beta)


def kernel(input_ids, token_type_ids, word_table, pos_table, type_table,
           gamma, beta):
    return _run(input_ids.astype(jnp.int32), token_type_ids.astype(jnp.int32),
                word_table, pos_table, type_table, gamma, beta)


# speedup vs baseline: 3.2941x; 3.2941x over previous
import functools
import jax
import jax.numpy as jnp
from jax import lax
from jax.experimental import pallas as pl
from jax.experimental.pallas import tpu as pltpu
from jax.experimental.pallas import tpu_sc as plsc

VOCAB = 30522
HIDDEN = 768
MAX_POS = 512
TYPE_VOCAB = 16
BATCH = 128
SEQ = 512

NW = 32
TOK = BATCH * SEQ
TPW = TOK // NW
CH = 64
NCH = TPW // CH
INV_H = 1.0 / HIDDEN
EPS = 1e-12


def _sc_gather_body(ids_hbm, word_hbm, out_hbm,
                    idx0, idx1, buf0, buf1, gsem0, gsem1, osem0, osem1):
    wid = lax.axis_index("s") * 2 + lax.axis_index("c")
    base = wid * TPW
    idx = (idx0, idx1)
    buf = (buf0, buf1)
    gsem = (gsem0, gsem1)
    osem = (osem0, osem1)

    def fire(c, slot):
        pltpu.sync_copy(ids_hbm.at[pl.ds(base + c * CH, CH)], idx[slot])
        pltpu.async_copy(word_hbm.at[idx[slot]], buf[slot], gsem[slot])

    def wait_gather(slot):
        pltpu.make_async_copy(word_hbm.at[idx[slot]], buf[slot],
                              gsem[slot]).wait()

    def start_out(c, slot):
        pltpu.async_copy(buf[slot],
                         out_hbm.at[pl.ds(base + c * CH, CH), :], osem[slot])

    def wait_out(c, slot):
        pltpu.make_async_copy(buf[slot],
                              out_hbm.at[pl.ds(base + c * CH, CH), :],
                              osem[slot]).wait()

    fire(0, 0)
    for c in range(NCH):
        slot = c % 2
        nslot = 1 - slot
        wait_gather(slot)
        if c + 1 < NCH:
            if c >= 1:
                wait_out(c - 1, nslot)
            fire(c + 1, nslot)
        start_out(c, slot)
    wait_out(NCH - 2, (NCH - 2) % 2)
    wait_out(NCH - 1, (NCH - 1) % 2)


def _sc_gather(ids_flat, word_table):
    mesh = plsc.VectorSubcoreMesh(core_axis_name="c", subcore_axis_name="s")
    f = pl.kernel(
        _sc_gather_body,
        out_type=jax.ShapeDtypeStruct((TOK, HIDDEN), jnp.float32),
        mesh=mesh,
        compiler_params=pltpu.CompilerParams(needs_layout_passes=False),
        scratch_types=[
            pltpu.VMEM((CH,), jnp.int32),
            pltpu.VMEM((CH,), jnp.int32),
            pltpu.VMEM((CH, HIDDEN), jnp.float32),
            pltpu.VMEM((CH, HIDDEN), jnp.float32),
            pltpu.SemaphoreType.DMA,
            pltpu.SemaphoreType.DMA,
            pltpu.SemaphoreType.DMA,
            pltpu.SemaphoreType.DMA,
        ],
    )
    return f(ids_flat, word_table)


def _tc_body(tt_ref, w_ref, pos_ref, type_ref, gam_ref, bet_ref, out_ref):
    w = w_ref[...]
    tt = tt_ref[0, 0, :]
    onehot = (tt[:, None] ==
              lax.broadcasted_iota(jnp.int32, (SEQ, TYPE_VOCAB), 1)
              ).astype(jnp.float32)
    temb = jnp.dot(onehot, type_ref[...],
                   preferred_element_type=jnp.float32,
                   precision=lax.Precision.HIGHEST)
    v = w + pos_ref[...] + temb
    mean = jnp.mean(v, axis=-1, keepdims=True)
    c = v - mean
    var = jnp.mean(c * c, axis=-1, keepdims=True)
    normed = c * lax.rsqrt(var + EPS)
    out_ref[...] = normed * gam_ref[...] + bet_ref[...]


def _tc_stage(token_type_ids, gathered, pos_table, type_table, gamma, beta):
    gamma2 = gamma.reshape(1, HIDDEN)
    beta2 = beta.reshape(1, HIDDEN)
    tt3 = token_type_ids.reshape(BATCH, 1, SEQ)
    out = pl.pallas_call(
        _tc_body,
        grid=(BATCH,),
        in_specs=[
            pl.BlockSpec((1, 1, SEQ), lambda b: (b, 0, 0)),
            pl.BlockSpec((SEQ, HIDDEN), lambda b: (b, 0)),
            pl.BlockSpec((MAX_POS, HIDDEN), lambda b: (0, 0)),
            pl.BlockSpec((TYPE_VOCAB, HIDDEN), lambda b: (0, 0)),
            pl.BlockSpec((1, HIDDEN), lambda b: (0, 0)),
            pl.BlockSpec((1, HIDDEN), lambda b: (0, 0)),
        ],
        out_specs=pl.BlockSpec((SEQ, HIDDEN), lambda b: (b, 0)),
        out_shape=jax.ShapeDtypeStruct((TOK, HIDDEN), jnp.float32),
    )(tt3, gathered, pos_table, type_table, gamma2, beta2)
    return out.reshape(BATCH, SEQ, HIDDEN)


@jax.jit
def _run(input_ids, token_type_ids, word_table, pos_table, type_table,
         gamma, beta):
    gathered = _sc_gather(input_ids.reshape(TOK), word_table)
    return _tc_stage(token_type_ids, gathered, pos_table, type_table,
                     gamma, beta)


def kernel(input_ids, token_type_ids, word_table, pos_table, type_table,
           gamma, beta):
    return _run(input_ids.astype(jnp.int32), token_type_ids.astype(jnp.int32),
                word_table, pos_table, type_table, gamma, beta)


# speedup vs baseline: 3.3042x; 1.0031x over previous
import functools
import jax
import jax.numpy as jnp
from jax import lax
from jax.experimental import pallas as pl
from jax.experimental.pallas import tpu as pltpu
from jax.experimental.pallas import tpu_sc as plsc

VOCAB = 30522
HIDDEN = 768
MAX_POS = 512
TYPE_VOCAB = 16
BATCH = 128
SEQ = 512

NW = 32
TOK = BATCH * SEQ
TPW = TOK // NW
CH = 64
NCH = TPW // CH
INV_H = 1.0 / HIDDEN
EPS = 1e-12


def _sc_gather_body(ids_hbm, word_hbm, out_hbm,
                    idx0, idx1, buf0, buf1, gsem0, gsem1, osem0, osem1):
    wid = lax.axis_index("s") * 2 + lax.axis_index("c")
    base = wid * TPW
    idx = (idx0, idx1)
    buf = (buf0, buf1)
    gsem = (gsem0, gsem1)
    osem = (osem0, osem1)

    def fire(c, slot):
        pltpu.sync_copy(ids_hbm.at[pl.ds(base + c * CH, CH)], idx[slot])
        pltpu.async_copy(word_hbm.at[idx[slot]], buf[slot], gsem[slot])

    def wait_gather(slot):
        pltpu.make_async_copy(word_hbm.at[idx[slot]], buf[slot],
                              gsem[slot]).wait()

    def start_out(c, slot):
        pltpu.async_copy(buf[slot],
                         out_hbm.at[pl.ds(base + c * CH, CH), :], osem[slot])

    def wait_out(c, slot):
        pltpu.make_async_copy(buf[slot],
                              out_hbm.at[pl.ds(base + c * CH, CH), :],
                              osem[slot]).wait()

    fire(0, 0)
    for c in range(NCH):
        slot = c % 2
        nslot = 1 - slot
        wait_gather(slot)
        if c + 1 < NCH:
            if c >= 1:
                wait_out(c - 1, nslot)
            fire(c + 1, nslot)
        start_out(c, slot)
    wait_out(NCH - 2, (NCH - 2) % 2)
    wait_out(NCH - 1, (NCH - 1) % 2)


def _sc_gather(ids_flat, word_table):
    mesh = plsc.VectorSubcoreMesh(core_axis_name="c", subcore_axis_name="s")
    f = pl.kernel(
        _sc_gather_body,
        out_type=jax.ShapeDtypeStruct((TOK, HIDDEN), jnp.float32),
        mesh=mesh,
        compiler_params=pltpu.CompilerParams(needs_layout_passes=False),
        scratch_types=[
            pltpu.VMEM((CH,), jnp.int32),
            pltpu.VMEM((CH,), jnp.int32),
            pltpu.VMEM((CH, HIDDEN), jnp.float32),
            pltpu.VMEM((CH, HIDDEN), jnp.float32),
            pltpu.SemaphoreType.DMA,
            pltpu.SemaphoreType.DMA,
            pltpu.SemaphoreType.DMA,
            pltpu.SemaphoreType.DMA,
        ],
    )
    return f(ids_flat, word_table)


def _tc_body(tt_ref, w_ref, pos_ref, type_ref, gam_ref, bet_ref, out_ref):
    w = w_ref[...]
    tt = tt_ref[0, 0, :]
    onehot = (tt[:, None] ==
              lax.broadcasted_iota(jnp.int32, (SEQ, TYPE_VOCAB), 1)
              ).astype(jnp.float32)
    temb = jnp.dot(onehot, type_ref[...],
                   preferred_element_type=jnp.float32,
                   precision=lax.Precision.HIGHEST)
    v = w + pos_ref[...] + temb
    mean = jnp.mean(v, axis=-1, keepdims=True)
    sq = jnp.mean(v * v, axis=-1, keepdims=True)
    rstd = lax.rsqrt(sq - mean * mean + EPS)
    out_ref[...] = (v - mean) * rstd * gam_ref[...] + bet_ref[...]


def _tc_stage(token_type_ids, gathered, pos_table, type_table, gamma, beta):
    gamma2 = gamma.reshape(1, HIDDEN)
    beta2 = beta.reshape(1, HIDDEN)
    tt3 = token_type_ids.reshape(BATCH, 1, SEQ)
    out = pl.pallas_call(
        _tc_body,
        grid=(BATCH,),
        in_specs=[
            pl.BlockSpec((1, 1, SEQ), lambda b: (b, 0, 0)),
            pl.BlockSpec((SEQ, HIDDEN), lambda b: (b, 0)),
            pl.BlockSpec((MAX_POS, HIDDEN), lambda b: (0, 0)),
            pl.BlockSpec((TYPE_VOCAB, HIDDEN), lambda b: (0, 0)),
            pl.BlockSpec((1, HIDDEN), lambda b: (0, 0)),
            pl.BlockSpec((1, HIDDEN), lambda b: (0, 0)),
        ],
        out_specs=pl.BlockSpec((SEQ, HIDDEN), lambda b: (b, 0)),
        out_shape=jax.ShapeDtypeStruct((TOK, HIDDEN), jnp.float32),
    )(tt3, gathered, pos_table, type_table, gamma2, beta2)
    return out.reshape(BATCH, SEQ, HIDDEN)


@jax.jit
def _run(input_ids, token_type_ids, word_table, pos_table, type_table,
         gamma, beta):
    gathered = _sc_gather(input_ids.reshape(TOK), word_table)
    return _tc_stage(token_type_ids, gathered, pos_table, type_table,
                     gamma, beta)


def kernel(input_ids, token_type_ids, word_table, pos_table, type_table,
           gamma, beta):
    return _run(input_ids.astype(jnp.int32), token_type_ids.astype(jnp.int32),
                word_table, pos_table, type_table, gamma, beta)
